# baseline (speedup 1.0000x reference)
import functools

import jax
import jax.numpy as jnp
from jax import lax
from jax.experimental import pallas as pl
from jax.experimental.pallas import tpu as pltpu
from jax.experimental.pallas import tpu_sc as plsc

N = 10000
E = 320000
D = 128
H = 128
C = 40
G = 16
P = 0.2

NP_ = 10240
EP = 327680
NC = 2
NS = 16
NW = NC * NS
ROWS_PER_TILE = NP_ // NS
EROWS = EP // 128
EROWS_PER_TILE = EROWS // NW
ICHUNK = 16
NCHUNK = EROWS_PER_TILE // ICHUNK
BLK = 256
NBLK = NP_ // BLK

_mesh = plsc.VectorSubcoreMesh(core_axis_name="c", subcore_axis_name="s")


@functools.partial(
    pl.kernel,
    out_type=jax.ShapeDtypeStruct((2 * NP_, 128), jnp.float32),
    mesh=_mesh,
    scratch_types=[
        pltpu.VMEM((EROWS_PER_TILE, 128), jnp.int32),
        pltpu.VMEM((128, 128), jnp.float32),
        pltpu.VMEM_SHARED((NP_, 128), jnp.float32),
        pltpu.SemaphoreType.DMA,
        pltpu.SemaphoreType.DMA,
    ],
)
def _sc_degree(dst_hbm, ones_hbm, zeros_hbm, out_hbm, didx, ones_v, cnt,
               sem0, sem1):
    c = lax.axis_index("c")
    s = lax.axis_index("s")
    wid = c * NS + s
    pltpu.sync_copy(dst_hbm.at[pl.ds(wid * EROWS_PER_TILE, EROWS_PER_TILE)], didx)
    pltpu.sync_copy(ones_hbm, ones_v)
    pltpu.sync_copy(zeros_hbm, cnt.at[pl.ds(s * ROWS_PER_TILE, ROWS_PER_TILE)])
    plsc.subcore_barrier()

    def _body(j, _):
        ds_ = [pltpu.async_copy(ones_v, cnt.at[didx.at[8 * j + k]],
                                sem0 if k % 2 == 0 else sem1, add=True)
               for k in range(8)]
        for d in ds_:
            d.wait()
        return _

    lax.fori_loop(0, EROWS_PER_TILE // 8, _body, 0)
    plsc.subcore_barrier()
    pltpu.sync_copy(cnt.at[pl.ds(s * ROWS_PER_TILE, ROWS_PER_TILE)],
                    out_hbm.at[pl.ds(c * NP_ + s * ROWS_PER_TILE, ROWS_PER_TILE)])


@functools.partial(
    pl.kernel,
    out_type=jax.ShapeDtypeStruct((2 * NP_, 128), jnp.float32),
    mesh=_mesh,
    scratch_types=[
        pltpu.VMEM((ICHUNK, 128), jnp.int32),
        pltpu.VMEM((ICHUNK, 128), jnp.int32),
        pltpu.VMEM((ICHUNK, 128), jnp.int32),
        pltpu.VMEM((ICHUNK, 128), jnp.int32),
        pltpu.VMEM((128, 128), jnp.float32),
        pltpu.VMEM((128, 128), jnp.float32),
        pltpu.VMEM_SHARED((NP_, 128), jnp.float32),
        pltpu.SemaphoreType.DMA,
        pltpu.SemaphoreType.DMA,
        pltpu.SemaphoreType.DMA,
        pltpu.SemaphoreType.DMA,
        pltpu.SemaphoreType.DMA,
    ],
)
def _sc_scatter(xws_hbm, src_hbm, dst_hbm, zeros_hbm, s_hbm,
                sidx0, didx0, sidx1, didx1, rows0, rows1, acc,
                semi, sem0, sem1, ssem0, ssem1):
    c = lax.axis_index("c")
    s = lax.axis_index("s")
    wid = c * NS + s
    rbase = wid * EROWS_PER_TILE
    sbufs = (sidx0, sidx1)
    dbufs = (didx0, didx1)

    pltpu.async_copy(src_hbm.at[pl.ds(rbase, ICHUNK)], sidx0, semi)
    pltpu.async_copy(dst_hbm.at[pl.ds(rbase, ICHUNK)], didx0, semi)
    pltpu.sync_copy(zeros_hbm, acc.at[pl.ds(s * ROWS_PER_TILE, ROWS_PER_TILE)])
    plsc.subcore_barrier()

    for t in range(NCHUNK):
        sA, dA = sbufs[t % 2], dbufs[t % 2]
        sB, dB = sbufs[(t + 1) % 2], dbufs[(t + 1) % 2]
        if t + 1 < NCHUNK:
            nb = rbase + (t + 1) * ICHUNK
            pltpu.async_copy(src_hbm.at[pl.ds(nb, ICHUNK)], sB, semi)
            pltpu.async_copy(dst_hbm.at[pl.ds(nb, ICHUNK)], dB, semi)
        cb = rbase + t * ICHUNK
        pltpu.make_async_copy(src_hbm.at[pl.ds(cb, ICHUNK)], sA, semi).wait()
        pltpu.make_async_copy(dst_hbm.at[pl.ds(cb, ICHUNK)], dA, semi).wait()

        pltpu.async_copy(xws_hbm.at[sA.at[0]], rows0, sem0)
        pltpu.async_copy(xws_hbm.at[sA.at[1]], rows1, sem1)

        def _body(j, carry, sA=sA, dA=dA):
            pltpu.make_async_copy(xws_hbm.at[sA.at[2 * j]], rows0, sem0).wait()
            s0 = pltpu.async_copy(rows0, acc.at[dA.at[2 * j]], ssem0, add=True)
            pltpu.make_async_copy(xws_hbm.at[sA.at[2 * j + 1]], rows1, sem1).wait()
            s1 = pltpu.async_copy(rows1, acc.at[dA.at[2 * j + 1]], ssem1, add=True)
            s0.wait()

            @pl.when(j < ICHUNK // 2 - 1)
            def _pf0():
                pltpu.async_copy(xws_hbm.at[sA.at[2 * j + 2]], rows0, sem0)

            s1.wait()

            @pl.when(j < ICHUNK // 2 - 1)
            def _pf1():
                pltpu.async_copy(xws_hbm.at[sA.at[2 * j + 3]], rows1, sem1)

            return carry

        lax.fori_loop(0, ICHUNK // 2, _body, 0)

    plsc.subcore_barrier()
    pltpu.sync_copy(acc.at[pl.ds(s * ROWS_PER_TILE, ROWS_PER_TILE)],
                    s_hbm.at[pl.ds(c * NP_ + s * ROWS_PER_TILE, ROWS_PER_TILE)])


def _gate_body(batch_ref, x_ref, q_ref, w_ref, cnt_ref,
               h_ref, xw_ref, xws_ref, dinv_ref):
    deg = cnt_ref[0, :, 0:1] + cnt_ref[1, :, 0:1] + 1.0
    dinv = jnp.broadcast_to(lax.rsqrt(deg), (BLK, 128))
    gid = lax.broadcasted_iota(jnp.int32, (BLK, 128), 1)
    oh = (batch_ref[...] == gid).astype(jnp.float32)
    h = jnp.dot(oh, q_ref[...], preferred_element_type=jnp.float32) * x_ref[...]
    xw = jnp.dot(h, w_ref[...], preferred_element_type=jnp.float32)
    h_ref[...] = h
    xw_ref[...] = xw
    xws_ref[...] = xw * dinv
    dinv_ref[...] = dinv


def _tc_gate(batch_b, x, q_pad, w1, counts):
    blk = pl.BlockSpec((BLK, 128), lambda i: (i, 0))
    cblk = pl.BlockSpec((2, BLK, 128), lambda i: (0, i, 0))
    full = pl.BlockSpec((128, 128), lambda i: (0, 0))
    out = jax.ShapeDtypeStruct((NP_, 128), jnp.float32)
    return pl.pallas_call(
        _gate_body,
        grid=(NBLK,),
        in_specs=[blk, blk, full, full, cblk],
        out_specs=[blk, blk, blk, blk],
        out_shape=[out, out, out, out],
    )(batch_b, x, q_pad, w1, counts)


def _mid_body(s_ref, xwp_ref, hp_ref, dinv_ref, b_ref, w_ref,
              h_ref, xw_ref, xws_ref):
    dv = dinv_ref[...]
    agg = dv * (s_ref[0] + s_ref[1]) + dv * dv * xwp_ref[...] + b_ref[...]
    h = jnp.maximum(agg + hp_ref[...], 0.0)
    xw = jnp.dot(h, w_ref[...], preferred_element_type=jnp.float32)
    h_ref[...] = h
    xw_ref[...] = xw
    xws_ref[...] = xw * dv


def _tc_mid(s_pair, xw_prev, h_prev, dinv_b, b_row, w_next):
    blk = pl.BlockSpec((BLK, 128), lambda i: (i, 0))
    sblk = pl.BlockSpec((2, BLK, 128), lambda i: (0, i, 0))
    brow = pl.BlockSpec((1, 128), lambda i: (0, 0))
    full = pl.BlockSpec((128, 128), lambda i: (0, 0))
    out = jax.ShapeDtypeStruct((NP_, 128), jnp.float32)
    return pl.pallas_call(
        _mid_body,
        grid=(NBLK,),
        in_specs=[sblk, blk, blk, blk, brow, full],
        out_specs=[blk, blk, blk],
        out_shape=[out, out, out],
    )(s_pair, xw_prev, h_prev, dinv_b, b_row, w_next)


FBLK = 400


def _final_body(s_ref, xwp_ref, hp_ref, dinv_ref, b_ref, wc_ref, bc_ref,
                mask_ref, out_ref):
    dv = dinv_ref[...]
    agg = dv * (s_ref[0] + s_ref[1]) + dv * dv * xwp_ref[...] + b_ref[...]
    h4 = agg + hp_ref[...]
    logits = jnp.dot(h4, wc_ref[...], preferred_element_type=jnp.float32)
    out_ref[...] = (logits + bc_ref[...]) * mask_ref[...]


def _tc_final(s_pair, xw_prev, h_prev, dinv_b, b_row, wc, bc_row, mask):
    blk = pl.BlockSpec((FBLK, 128), lambda i: (i, 0))
    sblk = pl.BlockSpec((2, FBLK, 128), lambda i: (0, i, 0))
    brow = pl.BlockSpec((1, 128), lambda i: (0, 0))
    return pl.pallas_call(
        _final_body,
        grid=(N // FBLK,),
        in_specs=[sblk, blk, blk, blk, brow,
                  pl.BlockSpec((128, C), lambda i: (0, 0)),
                  pl.BlockSpec((1, C), lambda i: (0, 0)),
                  pl.BlockSpec((FBLK, C), lambda i: (i, 0))],
        out_specs=pl.BlockSpec((FBLK, C), lambda i: (i, 0)),
        out_shape=jax.ShapeDtypeStruct((N, C), jnp.float32),
    )(s_pair, xw_prev, h_prev, dinv_b, b_row, wc, bc_row, mask)


def kernel(x, query, batch, edge_index, W1, b1, W2, b2, W3, b3, W4, b4, Wc, bc):
    f32 = jnp.float32
    x_p = jnp.pad(x, ((0, NP_ - N), (0, 0)))
    batch_p = jnp.pad(batch, (0, NP_ - N))
    batch_b = jnp.broadcast_to(batch_p[:, None], (NP_, 128))
    q_pad = jnp.pad(query, ((0, 128 - G), (0, 0)))
    bc_row = bc.reshape(1, C)
    b1r, b2r, b3r, b4r = (b.reshape(1, 128) for b in (b1, b2, b3, b4))

    pad_idx = N + jnp.arange(EP - E, dtype=jnp.int32) % (NP_ - N)
    src = jnp.concatenate([edge_index[0], pad_idx]).reshape(EROWS, 128)
    dst = jnp.concatenate([edge_index[1], pad_idx]).reshape(EROWS, 128)
    zeros = jnp.zeros((ROWS_PER_TILE, 128), f32)

    keep = jax.random.bernoulli(jax.random.key(42), 1.0 - P, (N, C))
    maskf = jnp.where(keep, f32(1.0) / f32(1.0 - P), f32(0.0))

    ones128 = jnp.ones((128, 128), f32)
    counts = _sc_degree(dst, ones128, zeros).reshape(2, NP_, 128)

    h0, xw1, xws1, dinv_b = _tc_gate(batch_b, x_p, q_pad, W1, counts)
    s1 = _sc_scatter(xws1, src, dst, zeros).reshape(2, NP_, 128)
    h1, xw2, xws2 = _tc_mid(s1, xw1, h0, dinv_b, b1r, W2)
    s2 = _sc_scatter(xws2, src, dst, zeros).reshape(2, NP_, 128)
    h2, xw3, xws3 = _tc_mid(s2, xw2, h1, dinv_b, b2r, W3)
    s3 = _sc_scatter(xws3, src, dst, zeros).reshape(2, NP_, 128)
    h3, xw4, xws4 = _tc_mid(s3, xw3, h2, dinv_b, b3r, W4)
    s4 = _sc_scatter(xws4, src, dst, zeros).reshape(2, NP_, 128)
    return _tc_final(s4, xw4, h3, dinv_b, b4r, Wc, bc_row, maskf)


# speedup vs baseline: 1.1750x; 1.1750x over previous
import functools

import jax
import jax.numpy as jnp
from jax import lax
from jax.experimental import pallas as pl
from jax.experimental.pallas import tpu as pltpu
from jax.experimental.pallas import tpu_sc as plsc

N = 10000
E = 320000
D = 128
H = 128
C = 40
G = 16
P = 0.2

NP_ = 10240
EP = 327680
NC = 2
NS = 16
NW = NC * NS
ROWS_PER_TILE = NP_ // NS
EROWS = EP // 128
EROWS_PER_TILE = EROWS // NW
ICHUNK = 16
NCHUNK = EROWS_PER_TILE // ICHUNK
BLK = 256
NBLK = NP_ // BLK

_mesh = plsc.VectorSubcoreMesh(core_axis_name="c", subcore_axis_name="s")


@functools.partial(
    pl.kernel,
    out_type=jax.ShapeDtypeStruct((2 * NP_, 128), jnp.float32),
    mesh=_mesh,
    scratch_types=[
        pltpu.VMEM((EROWS_PER_TILE, 128), jnp.int32),
        pltpu.VMEM((128, 128), jnp.float32),
        pltpu.VMEM_SHARED((NP_, 128), jnp.float32),
        pltpu.SemaphoreType.DMA,
        pltpu.SemaphoreType.DMA,
    ],
)
def _sc_degree(dst_hbm, ones_hbm, zeros_hbm, out_hbm, didx, ones_v, cnt,
               sem0, sem1):
    c = lax.axis_index("c")
    s = lax.axis_index("s")
    wid = c * NS + s
    pltpu.sync_copy(dst_hbm.at[pl.ds(wid * EROWS_PER_TILE, EROWS_PER_TILE)], didx)
    pltpu.sync_copy(ones_hbm, ones_v)
    pltpu.sync_copy(zeros_hbm, cnt.at[pl.ds(s * ROWS_PER_TILE, ROWS_PER_TILE)])
    plsc.subcore_barrier()

    def _body(j, _):
        ds_ = [pltpu.async_copy(ones_v, cnt.at[didx.at[8 * j + k]],
                                sem0 if k % 2 == 0 else sem1, add=True)
               for k in range(8)]
        for d in ds_:
            d.wait()
        return _

    lax.fori_loop(0, EROWS_PER_TILE // 8, _body, 0)
    plsc.subcore_barrier()
    pltpu.sync_copy(cnt.at[pl.ds(s * ROWS_PER_TILE, ROWS_PER_TILE)],
                    out_hbm.at[pl.ds(c * NP_ + s * ROWS_PER_TILE, ROWS_PER_TILE)])


@functools.partial(
    pl.kernel,
    out_type=jax.ShapeDtypeStruct((2 * NP_, 128), jnp.float32),
    mesh=_mesh,
    scratch_types=[
        pltpu.VMEM((ICHUNK, 128), jnp.int32),
        pltpu.VMEM((ICHUNK, 128), jnp.int32),
        pltpu.VMEM((ICHUNK, 128), jnp.int32),
        pltpu.VMEM((ICHUNK, 128), jnp.int32),
        pltpu.VMEM((128, 128), jnp.float32),
        pltpu.VMEM((128, 128), jnp.float32),
        pltpu.VMEM_SHARED((NP_, 128), jnp.float32),
        pltpu.SemaphoreType.DMA,
        pltpu.SemaphoreType.DMA,
        pltpu.SemaphoreType.DMA,
        pltpu.SemaphoreType.DMA,
        pltpu.SemaphoreType.DMA,
    ],
)
def _sc_scatter(xws_hbm, src_hbm, dst_hbm, zeros_hbm, s_hbm,
                sidx0, didx0, sidx1, didx1, rows0, rows1, acc,
                semi, sem0, sem1, ssem0, ssem1):
    c = lax.axis_index("c")
    s = lax.axis_index("s")
    wid = c * NS + s
    rbase = wid * EROWS_PER_TILE
    sbufs = (sidx0, sidx1)
    dbufs = (didx0, didx1)

    pltpu.async_copy(src_hbm.at[pl.ds(rbase, ICHUNK)], sidx0, semi)
    pltpu.async_copy(dst_hbm.at[pl.ds(rbase, ICHUNK)], didx0, semi)
    pltpu.sync_copy(zeros_hbm, acc.at[pl.ds(s * ROWS_PER_TILE, ROWS_PER_TILE)])
    plsc.subcore_barrier()

    for t in range(NCHUNK):
        sA, dA = sbufs[t % 2], dbufs[t % 2]
        sB, dB = sbufs[(t + 1) % 2], dbufs[(t + 1) % 2]
        if t + 1 < NCHUNK:
            nb = rbase + (t + 1) * ICHUNK
            pltpu.async_copy(src_hbm.at[pl.ds(nb, ICHUNK)], sB, semi)
            pltpu.async_copy(dst_hbm.at[pl.ds(nb, ICHUNK)], dB, semi)
        cb = rbase + t * ICHUNK
        pltpu.make_async_copy(src_hbm.at[pl.ds(cb, ICHUNK)], sA, semi).wait()
        pltpu.make_async_copy(dst_hbm.at[pl.ds(cb, ICHUNK)], dA, semi).wait()

        pltpu.async_copy(xws_hbm.at[sA.at[0]], rows0, sem0)
        pltpu.async_copy(xws_hbm.at[sA.at[1]], rows1, sem1)

        def _body(j, carry, sA=sA, dA=dA):
            pltpu.make_async_copy(xws_hbm.at[sA.at[2 * j]], rows0, sem0).wait()
            pltpu.sync_copy(rows0, acc.at[dA.at[2 * j]], add=True)

            @pl.when(j < ICHUNK // 2 - 1)
            def _pf0():
                pltpu.async_copy(xws_hbm.at[sA.at[2 * j + 2]], rows0, sem0)

            pltpu.make_async_copy(xws_hbm.at[sA.at[2 * j + 1]], rows1, sem1).wait()
            pltpu.sync_copy(rows1, acc.at[dA.at[2 * j + 1]], add=True)

            @pl.when(j < ICHUNK // 2 - 1)
            def _pf1():
                pltpu.async_copy(xws_hbm.at[sA.at[2 * j + 3]], rows1, sem1)

            return carry

        lax.fori_loop(0, ICHUNK // 2, _body, 0)

    plsc.subcore_barrier()
    pltpu.sync_copy(acc.at[pl.ds(s * ROWS_PER_TILE, ROWS_PER_TILE)],
                    s_hbm.at[pl.ds(c * NP_ + s * ROWS_PER_TILE, ROWS_PER_TILE)])


def _gate_body(batch_ref, x_ref, q_ref, w_ref, cnt_ref,
               h_ref, xw_ref, xws_ref, dinv_ref):
    deg = cnt_ref[0, :, 0:1] + cnt_ref[1, :, 0:1] + 1.0
    dinv = jnp.broadcast_to(lax.rsqrt(deg), (BLK, 128))
    gid = lax.broadcasted_iota(jnp.int32, (BLK, 128), 1)
    oh = (batch_ref[...] == gid).astype(jnp.float32)
    h = jnp.dot(oh, q_ref[...], preferred_element_type=jnp.float32) * x_ref[...]
    xw = jnp.dot(h, w_ref[...], preferred_element_type=jnp.float32)
    h_ref[...] = h
    xw_ref[...] = xw
    xws_ref[...] = xw * dinv
    dinv_ref[...] = dinv


def _tc_gate(batch_b, x, q_pad, w1, counts):
    blk = pl.BlockSpec((BLK, 128), lambda i: (i, 0))
    cblk = pl.BlockSpec((2, BLK, 128), lambda i: (0, i, 0))
    full = pl.BlockSpec((128, 128), lambda i: (0, 0))
    out = jax.ShapeDtypeStruct((NP_, 128), jnp.float32)
    return pl.pallas_call(
        _gate_body,
        grid=(NBLK,),
        in_specs=[blk, blk, full, full, cblk],
        out_specs=[blk, blk, blk, blk],
        out_shape=[out, out, out, out],
    )(batch_b, x, q_pad, w1, counts)


def _mid_body(s_ref, xwp_ref, hp_ref, dinv_ref, b_ref, w_ref,
              h_ref, xw_ref, xws_ref):
    dv = dinv_ref[...]
    agg = dv * (s_ref[0] + s_ref[1]) + dv * dv * xwp_ref[...] + b_ref[...]
    h = jnp.maximum(agg + hp_ref[...], 0.0)
    xw = jnp.dot(h, w_ref[...], preferred_element_type=jnp.float32)
    h_ref[...] = h
    xw_ref[...] = xw
    xws_ref[...] = xw * dv


def _tc_mid(s_pair, xw_prev, h_prev, dinv_b, b_row, w_next):
    blk = pl.BlockSpec((BLK, 128), lambda i: (i, 0))
    sblk = pl.BlockSpec((2, BLK, 128), lambda i: (0, i, 0))
    brow = pl.BlockSpec((1, 128), lambda i: (0, 0))
    full = pl.BlockSpec((128, 128), lambda i: (0, 0))
    out = jax.ShapeDtypeStruct((NP_, 128), jnp.float32)
    return pl.pallas_call(
        _mid_body,
        grid=(NBLK,),
        in_specs=[sblk, blk, blk, blk, brow, full],
        out_specs=[blk, blk, blk],
        out_shape=[out, out, out],
    )(s_pair, xw_prev, h_prev, dinv_b, b_row, w_next)


FBLK = 400


def _final_body(s_ref, xwp_ref, hp_ref, dinv_ref, b_ref, wc_ref, bc_ref,
                mask_ref, out_ref):
    dv = dinv_ref[...]
    agg = dv * (s_ref[0] + s_ref[1]) + dv * dv * xwp_ref[...] + b_ref[...]
    h4 = agg + hp_ref[...]
    logits = jnp.dot(h4, wc_ref[...], preferred_element_type=jnp.float32)
    out_ref[...] = (logits + bc_ref[...]) * mask_ref[...]


def _tc_final(s_pair, xw_prev, h_prev, dinv_b, b_row, wc, bc_row, mask):
    blk = pl.BlockSpec((FBLK, 128), lambda i: (i, 0))
    sblk = pl.BlockSpec((2, FBLK, 128), lambda i: (0, i, 0))
    brow = pl.BlockSpec((1, 128), lambda i: (0, 0))
    return pl.pallas_call(
        _final_body,
        grid=(N // FBLK,),
        in_specs=[sblk, blk, blk, blk, brow,
                  pl.BlockSpec((128, C), lambda i: (0, 0)),
                  pl.BlockSpec((1, C), lambda i: (0, 0)),
                  pl.BlockSpec((FBLK, C), lambda i: (i, 0))],
        out_specs=pl.BlockSpec((FBLK, C), lambda i: (i, 0)),
        out_shape=jax.ShapeDtypeStruct((N, C), jnp.float32),
    )(s_pair, xw_prev, h_prev, dinv_b, b_row, wc, bc_row, mask)


def kernel(x, query, batch, edge_index, W1, b1, W2, b2, W3, b3, W4, b4, Wc, bc):
    f32 = jnp.float32
    x_p = jnp.pad(x, ((0, NP_ - N), (0, 0)))
    batch_p = jnp.pad(batch, (0, NP_ - N))
    batch_b = jnp.broadcast_to(batch_p[:, None], (NP_, 128))
    q_pad = jnp.pad(query, ((0, 128 - G), (0, 0)))
    bc_row = bc.reshape(1, C)
    b1r, b2r, b3r, b4r = (b.reshape(1, 128) for b in (b1, b2, b3, b4))

    pad_idx = N + jnp.arange(EP - E, dtype=jnp.int32) % (NP_ - N)
    src = jnp.concatenate([edge_index[0], pad_idx]).reshape(EROWS, 128)
    dst = jnp.concatenate([edge_index[1], pad_idx]).reshape(EROWS, 128)
    zeros = jnp.zeros((ROWS_PER_TILE, 128), f32)

    keep = jax.random.bernoulli(jax.random.key(42), 1.0 - P, (N, C))
    maskf = jnp.where(keep, f32(1.0) / f32(1.0 - P), f32(0.0))

    ones128 = jnp.ones((128, 128), f32)
    counts = _sc_degree(dst, ones128, zeros).reshape(2, NP_, 128)

    h0, xw1, xws1, dinv_b = _tc_gate(batch_b, x_p, q_pad, W1, counts)
    s1 = _sc_scatter(xws1, src, dst, zeros).reshape(2, NP_, 128)
    h1, xw2, xws2 = _tc_mid(s1, xw1, h0, dinv_b, b1r, W2)
    s2 = _sc_scatter(xws2, src, dst, zeros).reshape(2, NP_, 128)
    h2, xw3, xws3 = _tc_mid(s2, xw2, h1, dinv_b, b2r, W3)
    s3 = _sc_scatter(xws3, src, dst, zeros).reshape(2, NP_, 128)
    h3, xw4, xws4 = _tc_mid(s3, xw3, h2, dinv_b, b3r, W4)
    s4 = _sc_scatter(xws4, src, dst, zeros).reshape(2, NP_, 128)
    return _tc_final(s4, xw4, h3, dinv_b, b4r, Wc, bc_row, maskf)


# speedup vs baseline: 1.1930x; 1.0153x over previous
import functools

import jax
import jax.numpy as jnp
from jax import lax
from jax.experimental import pallas as pl
from jax.experimental.pallas import tpu as pltpu
from jax.experimental.pallas import tpu_sc as plsc

N = 10000
E = 320000
D = 128
H = 128
C = 40
G = 16
P = 0.2

NP_ = 10240
EP = 327680
NC = 2
NS = 16
NW = NC * NS
ROWS_PER_TILE = NP_ // NS
EROWS = EP // 128
EROWS_PER_TILE = EROWS // NW
ICHUNK = 16
NCHUNK = EROWS_PER_TILE // ICHUNK
BLK = 256
NBLK = NP_ // BLK

_mesh = plsc.VectorSubcoreMesh(core_axis_name="c", subcore_axis_name="s")


@functools.partial(
    pl.kernel,
    out_type=jax.ShapeDtypeStruct((2 * NP_, 128), jnp.float32),
    mesh=_mesh,
    scratch_types=[
        pltpu.VMEM((EROWS_PER_TILE, 128), jnp.int32),
        pltpu.VMEM((128, 128), jnp.float32),
        pltpu.VMEM_SHARED((NP_, 128), jnp.float32),
        pltpu.SemaphoreType.DMA,
        pltpu.SemaphoreType.DMA,
    ],
)
def _sc_degree(dst_hbm, ones_hbm, zeros_hbm, out_hbm, didx, ones_v, cnt,
               sem0, sem1):
    c = lax.axis_index("c")
    s = lax.axis_index("s")
    wid = c * NS + s
    pltpu.sync_copy(dst_hbm.at[pl.ds(wid * EROWS_PER_TILE, EROWS_PER_TILE)], didx)
    pltpu.sync_copy(ones_hbm, ones_v)
    pltpu.sync_copy(zeros_hbm, cnt.at[pl.ds(s * ROWS_PER_TILE, ROWS_PER_TILE)])
    plsc.subcore_barrier()

    def _body(j, _):
        ds_ = [pltpu.async_copy(ones_v, cnt.at[didx.at[8 * j + k]],
                                sem0 if k % 2 == 0 else sem1, add=True)
               for k in range(8)]
        for d in ds_:
            d.wait()
        return _

    lax.fori_loop(0, EROWS_PER_TILE // 8, _body, 0)
    plsc.subcore_barrier()
    pltpu.sync_copy(cnt.at[pl.ds(s * ROWS_PER_TILE, ROWS_PER_TILE)],
                    out_hbm.at[pl.ds(c * NP_ + s * ROWS_PER_TILE, ROWS_PER_TILE)])


@functools.partial(
    pl.kernel,
    out_type=jax.ShapeDtypeStruct((2 * NP_, 128), jnp.float32),
    mesh=_mesh,
    scratch_types=[
        pltpu.VMEM((ICHUNK, 128), jnp.int32),
        pltpu.VMEM((ICHUNK, 128), jnp.int32),
        pltpu.VMEM((ICHUNK, 128), jnp.int32),
        pltpu.VMEM((ICHUNK, 128), jnp.int32),
        pltpu.VMEM((128, 128), jnp.float32),
        pltpu.VMEM((128, 128), jnp.float32),
        pltpu.VMEM_SHARED((NP_, 128), jnp.float32),
        pltpu.SemaphoreType.DMA,
        pltpu.SemaphoreType.DMA,
        pltpu.SemaphoreType.DMA,
        pltpu.SemaphoreType.DMA,
        pltpu.SemaphoreType.DMA,
    ],
)
def _sc_scatter(xws_hbm, src_hbm, dst_hbm, zeros_hbm, s_hbm,
                sidx0, didx0, sidx1, didx1, rows0, rows1, acc,
                semi, sem0, sem1, ssem0, ssem1):
    c = lax.axis_index("c")
    s = lax.axis_index("s")
    wid = c * NS + s
    rbase = wid * EROWS_PER_TILE
    sbufs = (sidx0, sidx1)
    dbufs = (didx0, didx1)

    pltpu.async_copy(src_hbm.at[pl.ds(rbase, ICHUNK)], sidx0, semi)
    pltpu.async_copy(dst_hbm.at[pl.ds(rbase, ICHUNK)], didx0, semi)
    pltpu.sync_copy(zeros_hbm, acc.at[pl.ds(s * ROWS_PER_TILE, ROWS_PER_TILE)])
    plsc.subcore_barrier()

    for t in range(NCHUNK):
        sA, dA = sbufs[t % 2], dbufs[t % 2]
        sB, dB = sbufs[(t + 1) % 2], dbufs[(t + 1) % 2]
        if t + 1 < NCHUNK:
            nb = rbase + (t + 1) * ICHUNK
            pltpu.async_copy(src_hbm.at[pl.ds(nb, ICHUNK)], sB, semi)
            pltpu.async_copy(dst_hbm.at[pl.ds(nb, ICHUNK)], dB, semi)
        cb = rbase + t * ICHUNK
        pltpu.make_async_copy(src_hbm.at[pl.ds(cb, ICHUNK)], sA, semi).wait()
        pltpu.make_async_copy(dst_hbm.at[pl.ds(cb, ICHUNK)], dA, semi).wait()

        pltpu.async_copy(xws_hbm.at[sA.at[0]], rows0, sem0)
        pltpu.async_copy(xws_hbm.at[sA.at[1]], rows1, sem1)

        def _body(j, carry, sA=sA, dA=dA):
            pltpu.make_async_copy(xws_hbm.at[sA.at[2 * j]], rows0, sem0).wait()
            pltpu.sync_copy(rows0, acc.at[dA.at[2 * j]], add=True)

            @pl.when(j < ICHUNK // 2 - 1)
            def _pf0():
                pltpu.async_copy(xws_hbm.at[sA.at[2 * j + 2]], rows0, sem0)

            pltpu.make_async_copy(xws_hbm.at[sA.at[2 * j + 1]], rows1, sem1).wait()
            pltpu.sync_copy(rows1, acc.at[dA.at[2 * j + 1]], add=True)

            @pl.when(j < ICHUNK // 2 - 1)
            def _pf1():
                pltpu.async_copy(xws_hbm.at[sA.at[2 * j + 3]], rows1, sem1)

            return carry

        lax.fori_loop(0, ICHUNK // 2, _body, 0)

    plsc.subcore_barrier()
    pltpu.sync_copy(acc.at[pl.ds(s * ROWS_PER_TILE, ROWS_PER_TILE)],
                    s_hbm.at[pl.ds(c * NP_ + s * ROWS_PER_TILE, ROWS_PER_TILE)])


def _gate_pre_body(batch_ref, x_ref, q_ref, w_ref, h_ref, xw_ref):
    gid = lax.broadcasted_iota(jnp.int32, (BLK, 128), 1)
    oh = (batch_ref[...] == gid).astype(jnp.float32)
    h = jnp.dot(oh, q_ref[...], preferred_element_type=jnp.float32) * x_ref[...]
    h_ref[...] = h
    xw_ref[...] = jnp.dot(h, w_ref[...], preferred_element_type=jnp.float32)


def _tc_gate_pre(batch_b, x, q_pad, w1):
    blk = pl.BlockSpec((BLK, 128), lambda i: (i, 0))
    full = pl.BlockSpec((128, 128), lambda i: (0, 0))
    out = jax.ShapeDtypeStruct((NP_, 128), jnp.float32)
    return pl.pallas_call(
        _gate_pre_body,
        grid=(NBLK,),
        in_specs=[blk, blk, full, full],
        out_specs=[blk, blk],
        out_shape=[out, out],
    )(batch_b, x, q_pad, w1)


def _gate_post_body(xw_ref, cnt_ref, xws_ref, dinv_ref):
    deg = cnt_ref[0, :, 0:1] + cnt_ref[1, :, 0:1] + 1.0
    dinv = jnp.broadcast_to(lax.rsqrt(deg), (BLK, 128))
    xws_ref[...] = xw_ref[...] * dinv
    dinv_ref[...] = dinv


def _tc_gate_post(xw1, counts):
    blk = pl.BlockSpec((BLK, 128), lambda i: (i, 0))
    cblk = pl.BlockSpec((2, BLK, 128), lambda i: (0, i, 0))
    out = jax.ShapeDtypeStruct((NP_, 128), jnp.float32)
    return pl.pallas_call(
        _gate_post_body,
        grid=(NBLK,),
        in_specs=[blk, cblk],
        out_specs=[blk, blk],
        out_shape=[out, out],
    )(xw1, counts)


def _mid_body(s_ref, xwp_ref, hp_ref, dinv_ref, b_ref, w_ref,
              h_ref, xw_ref, xws_ref):
    dv = dinv_ref[...]
    agg = dv * (s_ref[0] + s_ref[1]) + dv * dv * xwp_ref[...] + b_ref[...]
    h = jnp.maximum(agg + hp_ref[...], 0.0)
    xw = jnp.dot(h, w_ref[...], preferred_element_type=jnp.float32)
    h_ref[...] = h
    xw_ref[...] = xw
    xws_ref[...] = xw * dv


def _tc_mid(s_pair, xw_prev, h_prev, dinv_b, b_row, w_next):
    blk = pl.BlockSpec((BLK, 128), lambda i: (i, 0))
    sblk = pl.BlockSpec((2, BLK, 128), lambda i: (0, i, 0))
    brow = pl.BlockSpec((1, 128), lambda i: (0, 0))
    full = pl.BlockSpec((128, 128), lambda i: (0, 0))
    out = jax.ShapeDtypeStruct((NP_, 128), jnp.float32)
    return pl.pallas_call(
        _mid_body,
        grid=(NBLK,),
        in_specs=[sblk, blk, blk, blk, brow, full],
        out_specs=[blk, blk, blk],
        out_shape=[out, out, out],
    )(s_pair, xw_prev, h_prev, dinv_b, b_row, w_next)


FBLK = 400


def _final_body(s_ref, xwp_ref, hp_ref, dinv_ref, b_ref, wc_ref, bc_ref,
                mask_ref, out_ref):
    dv = dinv_ref[...]
    agg = dv * (s_ref[0] + s_ref[1]) + dv * dv * xwp_ref[...] + b_ref[...]
    h4 = agg + hp_ref[...]
    logits = jnp.dot(h4, wc_ref[...], preferred_element_type=jnp.float32)
    out_ref[...] = (logits + bc_ref[...]) * mask_ref[...]


def _tc_final(s_pair, xw_prev, h_prev, dinv_b, b_row, wc, bc_row, mask):
    blk = pl.BlockSpec((FBLK, 128), lambda i: (i, 0))
    sblk = pl.BlockSpec((2, FBLK, 128), lambda i: (0, i, 0))
    brow = pl.BlockSpec((1, 128), lambda i: (0, 0))
    return pl.pallas_call(
        _final_body,
        grid=(N // FBLK,),
        in_specs=[sblk, blk, blk, blk, brow,
                  pl.BlockSpec((128, C), lambda i: (0, 0)),
                  pl.BlockSpec((1, C), lambda i: (0, 0)),
                  pl.BlockSpec((FBLK, C), lambda i: (i, 0))],
        out_specs=pl.BlockSpec((FBLK, C), lambda i: (i, 0)),
        out_shape=jax.ShapeDtypeStruct((N, C), jnp.float32),
    )(s_pair, xw_prev, h_prev, dinv_b, b_row, wc, bc_row, mask)


def kernel(x, query, batch, edge_index, W1, b1, W2, b2, W3, b3, W4, b4, Wc, bc):
    f32 = jnp.float32
    x_p = jnp.pad(x, ((0, NP_ - N), (0, 0)))
    batch_p = jnp.pad(batch, (0, NP_ - N))
    batch_b = jnp.broadcast_to(batch_p[:, None], (NP_, 128))
    q_pad = jnp.pad(query, ((0, 128 - G), (0, 0)))
    bc_row = bc.reshape(1, C)
    b1r, b2r, b3r, b4r = (b.reshape(1, 128) for b in (b1, b2, b3, b4))

    pad_idx = N + jnp.arange(EP - E, dtype=jnp.int32) % (NP_ - N)
    src = jnp.concatenate([edge_index[0], pad_idx]).reshape(EROWS, 128)
    dst = jnp.concatenate([edge_index[1], pad_idx]).reshape(EROWS, 128)
    zeros = jnp.zeros((ROWS_PER_TILE, 128), f32)

    keep = jax.random.bernoulli(jax.random.key(42), 1.0 - P, (N, C))
    maskf = jnp.where(keep, f32(1.0) / f32(1.0 - P), f32(0.0))

    ones128 = jnp.ones((128, 128), f32)
    counts = _sc_degree(dst, ones128, zeros).reshape(2, NP_, 128)

    h0, xw1 = _tc_gate_pre(batch_b, x_p, q_pad, W1)
    xws1, dinv_b = _tc_gate_post(xw1, counts)
    s1 = _sc_scatter(xws1, src, dst, zeros).reshape(2, NP_, 128)
    h1, xw2, xws2 = _tc_mid(s1, xw1, h0, dinv_b, b1r, W2)
    s2 = _sc_scatter(xws2, src, dst, zeros).reshape(2, NP_, 128)
    h2, xw3, xws3 = _tc_mid(s2, xw2, h1, dinv_b, b2r, W3)
    s3 = _sc_scatter(xws3, src, dst, zeros).reshape(2, NP_, 128)
    h3, xw4, xws4 = _tc_mid(s3, xw3, h2, dinv_b, b3r, W4)
    s4 = _sc_scatter(xws4, src, dst, zeros).reshape(2, NP_, 128)
    return _tc_final(s4, xw4, h3, dinv_b, b4r, Wc, bc_row, maskf)


# speedup vs baseline: 1.2678x; 1.0627x over previous
import functools

import jax
import jax.numpy as jnp
from jax import lax
from jax.experimental import pallas as pl
from jax.experimental.pallas import tpu as pltpu
from jax.experimental.pallas import tpu_sc as plsc

N = 10000
E = 320000
D = 128
H = 128
C = 40
G = 16
P = 0.2

NP_ = 10240
EP = 327680
NC = 2
NS = 16
NW = NC * NS
ROWS_PER_TILE = NP_ // NS
EROWS = EP // 128
EROWS_PER_TILE = EROWS // NW
ICHUNK = 16
NCHUNK = EROWS_PER_TILE // ICHUNK
BLK = 512
NBLK = NP_ // BLK

_mesh = plsc.VectorSubcoreMesh(core_axis_name="c", subcore_axis_name="s")


@functools.partial(
    pl.kernel,
    out_type=jax.ShapeDtypeStruct((2 * NP_, 128), jnp.float32),
    mesh=_mesh,
    scratch_types=[
        pltpu.VMEM((EROWS_PER_TILE, 128), jnp.int32),
        pltpu.VMEM((128, 128), jnp.float32),
        pltpu.VMEM_SHARED((NP_, 128), jnp.float32),
        pltpu.SemaphoreType.DMA,
        pltpu.SemaphoreType.DMA,
    ],
)
def _sc_degree(dst_hbm, ones_hbm, zeros_hbm, out_hbm, didx, ones_v, cnt,
               sem0, sem1):
    c = lax.axis_index("c")
    s = lax.axis_index("s")
    wid = c * NS + s
    pltpu.sync_copy(dst_hbm.at[pl.ds(wid * EROWS_PER_TILE, EROWS_PER_TILE)], didx)
    pltpu.sync_copy(ones_hbm, ones_v)
    pltpu.sync_copy(zeros_hbm, cnt.at[pl.ds(s * ROWS_PER_TILE, ROWS_PER_TILE)])
    plsc.subcore_barrier()

    def _body(j, _):
        ds_ = [pltpu.async_copy(ones_v, cnt.at[didx.at[8 * j + k]],
                                sem0 if k % 2 == 0 else sem1, add=True)
               for k in range(8)]
        for d in ds_:
            d.wait()
        return _

    lax.fori_loop(0, EROWS_PER_TILE // 8, _body, 0)
    plsc.subcore_barrier()
    pltpu.sync_copy(cnt.at[pl.ds(s * ROWS_PER_TILE, ROWS_PER_TILE)],
                    out_hbm.at[pl.ds(c * NP_ + s * ROWS_PER_TILE, ROWS_PER_TILE)])


@functools.partial(
    pl.kernel,
    out_type=jax.ShapeDtypeStruct((2 * NP_, 128), jnp.float32),
    mesh=_mesh,
    scratch_types=[
        pltpu.VMEM((ICHUNK, 128), jnp.int32),
        pltpu.VMEM((ICHUNK, 128), jnp.int32),
        pltpu.VMEM((ICHUNK, 128), jnp.int32),
        pltpu.VMEM((ICHUNK, 128), jnp.int32),
        pltpu.VMEM((128, 128), jnp.float32),
        pltpu.VMEM((128, 128), jnp.float32),
        pltpu.VMEM_SHARED((NP_, 128), jnp.float32),
        pltpu.SemaphoreType.DMA,
        pltpu.SemaphoreType.DMA,
        pltpu.SemaphoreType.DMA,
        pltpu.SemaphoreType.DMA,
        pltpu.SemaphoreType.DMA,
    ],
)
def _sc_scatter(xws_hbm, src_hbm, dst_hbm, zeros_hbm, s_hbm,
                sidx0, didx0, sidx1, didx1, rows0, rows1, acc,
                semi, sem0, sem1, ssem0, ssem1):
    c = lax.axis_index("c")
    s = lax.axis_index("s")
    wid = c * NS + s
    rbase = wid * EROWS_PER_TILE
    sbufs = (sidx0, sidx1)
    dbufs = (didx0, didx1)

    pltpu.async_copy(src_hbm.at[pl.ds(rbase, ICHUNK)], sidx0, semi)
    pltpu.async_copy(dst_hbm.at[pl.ds(rbase, ICHUNK)], didx0, semi)
    pltpu.sync_copy(zeros_hbm, acc.at[pl.ds(s * ROWS_PER_TILE, ROWS_PER_TILE)])
    plsc.subcore_barrier()

    for t in range(NCHUNK):
        sA, dA = sbufs[t % 2], dbufs[t % 2]
        sB, dB = sbufs[(t + 1) % 2], dbufs[(t + 1) % 2]
        if t + 1 < NCHUNK:
            nb = rbase + (t + 1) * ICHUNK
            pltpu.async_copy(src_hbm.at[pl.ds(nb, ICHUNK)], sB, semi)
            pltpu.async_copy(dst_hbm.at[pl.ds(nb, ICHUNK)], dB, semi)
        cb = rbase + t * ICHUNK
        pltpu.make_async_copy(src_hbm.at[pl.ds(cb, ICHUNK)], sA, semi).wait()
        pltpu.make_async_copy(dst_hbm.at[pl.ds(cb, ICHUNK)], dA, semi).wait()

        pltpu.async_copy(xws_hbm.at[sA.at[0]], rows0, sem0)
        pltpu.async_copy(xws_hbm.at[sA.at[1]], rows1, sem1)

        def _body(j, carry, sA=sA, dA=dA):
            pltpu.make_async_copy(xws_hbm.at[sA.at[2 * j]], rows0, sem0).wait()
            pltpu.sync_copy(rows0, acc.at[dA.at[2 * j]], add=True)

            @pl.when(j < ICHUNK // 2 - 1)
            def _pf0():
                pltpu.async_copy(xws_hbm.at[sA.at[2 * j + 2]], rows0, sem0)

            pltpu.make_async_copy(xws_hbm.at[sA.at[2 * j + 1]], rows1, sem1).wait()
            pltpu.sync_copy(rows1, acc.at[dA.at[2 * j + 1]], add=True)

            @pl.when(j < ICHUNK // 2 - 1)
            def _pf1():
                pltpu.async_copy(xws_hbm.at[sA.at[2 * j + 3]], rows1, sem1)

            return carry

        lax.fori_loop(0, ICHUNK // 2, _body, 0)

    plsc.subcore_barrier()
    pltpu.sync_copy(acc.at[pl.ds(s * ROWS_PER_TILE, ROWS_PER_TILE)],
                    s_hbm.at[pl.ds(c * NP_ + s * ROWS_PER_TILE, ROWS_PER_TILE)])


def _gate_pre_body(batch_ref, x_ref, q_ref, w_ref, h_ref, xw_ref):
    gid = lax.broadcasted_iota(jnp.int32, (BLK, 128), 1)
    oh = (batch_ref[...] == gid).astype(jnp.float32)
    h = jnp.dot(oh, q_ref[...], preferred_element_type=jnp.float32) * x_ref[...]
    h_ref[...] = h
    xw_ref[...] = jnp.dot(h, w_ref[...], preferred_element_type=jnp.float32)


def _tc_gate_pre(batch_b, x, q_pad, w1):
    blk = pl.BlockSpec((BLK, 128), lambda i: (i, 0))
    full = pl.BlockSpec((128, 128), lambda i: (0, 0))
    out = jax.ShapeDtypeStruct((NP_, 128), jnp.float32)
    return pl.pallas_call(
        _gate_pre_body,
        grid=(NBLK,),
        in_specs=[blk, blk, full, full],
        out_specs=[blk, blk],
        out_shape=[out, out],
    )(batch_b, x, q_pad, w1)


def _gate_post_body(xw_ref, cnt_ref, xws_ref, dinv_ref):
    deg = cnt_ref[0, :, 0:1] + cnt_ref[1, :, 0:1] + 1.0
    dinv = jnp.broadcast_to(lax.rsqrt(deg), (BLK, 128))
    xws_ref[...] = xw_ref[...] * dinv
    dinv_ref[...] = dinv


def _tc_gate_post(xw1, counts):
    blk = pl.BlockSpec((BLK, 128), lambda i: (i, 0))
    cblk = pl.BlockSpec((2, BLK, 128), lambda i: (0, i, 0))
    out = jax.ShapeDtypeStruct((NP_, 128), jnp.float32)
    return pl.pallas_call(
        _gate_post_body,
        grid=(NBLK,),
        in_specs=[blk, cblk],
        out_specs=[blk, blk],
        out_shape=[out, out],
    )(xw1, counts)


def _mid_body(s_ref, xwp_ref, hp_ref, dinv_ref, b_ref, w_ref,
              h_ref, xw_ref, xws_ref):
    dv = dinv_ref[...]
    agg = dv * (s_ref[0] + s_ref[1]) + dv * dv * xwp_ref[...] + b_ref[...]
    h = jnp.maximum(agg + hp_ref[...], 0.0)
    xw = jnp.dot(h, w_ref[...], preferred_element_type=jnp.float32)
    h_ref[...] = h
    xw_ref[...] = xw
    xws_ref[...] = xw * dv


def _tc_mid(s_pair, xw_prev, h_prev, dinv_b, b_row, w_next):
    blk = pl.BlockSpec((BLK, 128), lambda i: (i, 0))
    sblk = pl.BlockSpec((2, BLK, 128), lambda i: (0, i, 0))
    brow = pl.BlockSpec((1, 128), lambda i: (0, 0))
    full = pl.BlockSpec((128, 128), lambda i: (0, 0))
    out = jax.ShapeDtypeStruct((NP_, 128), jnp.float32)
    return pl.pallas_call(
        _mid_body,
        grid=(NBLK,),
        in_specs=[sblk, blk, blk, blk, brow, full],
        out_specs=[blk, blk, blk],
        out_shape=[out, out, out],
    )(s_pair, xw_prev, h_prev, dinv_b, b_row, w_next)


FBLK = 400


def _final_body(s_ref, xwp_ref, hp_ref, dinv_ref, b_ref, wc_ref, bc_ref,
                mask_ref, out_ref):
    dv = dinv_ref[...]
    agg = dv * (s_ref[0] + s_ref[1]) + dv * dv * xwp_ref[...] + b_ref[...]
    h4 = agg + hp_ref[...]
    logits = jnp.dot(h4, wc_ref[...], preferred_element_type=jnp.float32)
    out_ref[...] = (logits + bc_ref[...]) * mask_ref[...]


def _tc_final(s_pair, xw_prev, h_prev, dinv_b, b_row, wc, bc_row, mask):
    blk = pl.BlockSpec((FBLK, 128), lambda i: (i, 0))
    sblk = pl.BlockSpec((2, FBLK, 128), lambda i: (0, i, 0))
    brow = pl.BlockSpec((1, 128), lambda i: (0, 0))
    return pl.pallas_call(
        _final_body,
        grid=(N // FBLK,),
        in_specs=[sblk, blk, blk, blk, brow,
                  pl.BlockSpec((128, C), lambda i: (0, 0)),
                  pl.BlockSpec((1, C), lambda i: (0, 0)),
                  pl.BlockSpec((FBLK, C), lambda i: (i, 0))],
        out_specs=pl.BlockSpec((FBLK, C), lambda i: (i, 0)),
        out_shape=jax.ShapeDtypeStruct((N, C), jnp.float32),
    )(s_pair, xw_prev, h_prev, dinv_b, b_row, wc, bc_row, mask)


def kernel(x, query, batch, edge_index, W1, b1, W2, b2, W3, b3, W4, b4, Wc, bc):
    f32 = jnp.float32
    x_p = jnp.pad(x, ((0, NP_ - N), (0, 0)))
    batch_p = jnp.pad(batch, (0, NP_ - N))
    batch_b = jnp.broadcast_to(batch_p[:, None], (NP_, 128))
    q_pad = jnp.pad(query, ((0, 128 - G), (0, 0)))
    bc_row = bc.reshape(1, C)
    b1r, b2r, b3r, b4r = (b.reshape(1, 128) for b in (b1, b2, b3, b4))

    pad_idx = N + jnp.arange(EP - E, dtype=jnp.int32) % (NP_ - N)
    src = jnp.concatenate([edge_index[0], pad_idx]).reshape(EROWS, 128)
    dst = jnp.concatenate([edge_index[1], pad_idx]).reshape(EROWS, 128)
    zeros = jnp.zeros((ROWS_PER_TILE, 128), f32)

    keep = jax.random.bernoulli(jax.random.key(42), 1.0 - P, (N, C))
    maskf = jnp.where(keep, f32(1.0) / f32(1.0 - P), f32(0.0))

    ones128 = jnp.ones((128, 128), f32)
    counts = _sc_degree(dst, ones128, zeros).reshape(2, NP_, 128)

    h0, xw1 = _tc_gate_pre(batch_b, x_p, q_pad, W1)
    xws1, dinv_b = _tc_gate_post(xw1, counts)
    s1 = _sc_scatter(xws1, src, dst, zeros).reshape(2, NP_, 128)
    h1, xw2, xws2 = _tc_mid(s1, xw1, h0, dinv_b, b1r, W2)
    s2 = _sc_scatter(xws2, src, dst, zeros).reshape(2, NP_, 128)
    h2, xw3, xws3 = _tc_mid(s2, xw2, h1, dinv_b, b2r, W3)
    s3 = _sc_scatter(xws3, src, dst, zeros).reshape(2, NP_, 128)
    h3, xw4, xws4 = _tc_mid(s3, xw3, h2, dinv_b, b3r, W4)
    s4 = _sc_scatter(xws4, src, dst, zeros).reshape(2, NP_, 128)
    return _tc_final(s4, xw4, h3, dinv_b, b4r, Wc, bc_row, maskf)


# speedup vs baseline: 1.3238x; 1.0442x over previous
import functools

import jax
import jax.numpy as jnp
from jax import lax
from jax.experimental import pallas as pl
from jax.experimental.pallas import tpu as pltpu
from jax.experimental.pallas import tpu_sc as plsc

N = 10000
E = 320000
D = 128
H = 128
C = 40
G = 16
P = 0.2

NP_ = 10240
EP = 327680
NC = 2
NS = 16
NW = NC * NS
ROWS_PER_TILE = NP_ // NS
EROWS = EP // 128
EROWS_PER_TILE = EROWS // NW
ICHUNK = 16
NCHUNK = EROWS_PER_TILE // ICHUNK
BLK = 1024
NBLK = NP_ // BLK

_mesh = plsc.VectorSubcoreMesh(core_axis_name="c", subcore_axis_name="s")


@functools.partial(
    pl.kernel,
    out_type=jax.ShapeDtypeStruct((2 * NP_, 128), jnp.float32),
    mesh=_mesh,
    scratch_types=[
        pltpu.VMEM((EROWS_PER_TILE, 128), jnp.int32),
        pltpu.VMEM((128, 128), jnp.float32),
        pltpu.VMEM_SHARED((NP_, 128), jnp.float32),
        pltpu.SemaphoreType.DMA,
        pltpu.SemaphoreType.DMA,
    ],
)
def _sc_degree(dst_hbm, ones_hbm, zeros_hbm, out_hbm, didx, ones_v, cnt,
               sem0, sem1):
    c = lax.axis_index("c")
    s = lax.axis_index("s")
    wid = c * NS + s
    pltpu.sync_copy(dst_hbm.at[pl.ds(wid * EROWS_PER_TILE, EROWS_PER_TILE)], didx)
    pltpu.sync_copy(ones_hbm, ones_v)
    pltpu.sync_copy(zeros_hbm, cnt.at[pl.ds(s * ROWS_PER_TILE, ROWS_PER_TILE)])
    plsc.subcore_barrier()

    def _body(j, _):
        ds_ = [pltpu.async_copy(ones_v, cnt.at[didx.at[8 * j + k]],
                                sem0 if k % 2 == 0 else sem1, add=True)
               for k in range(8)]
        for d in ds_:
            d.wait()
        return _

    lax.fori_loop(0, EROWS_PER_TILE // 8, _body, 0)
    plsc.subcore_barrier()
    pltpu.sync_copy(cnt.at[pl.ds(s * ROWS_PER_TILE, ROWS_PER_TILE)],
                    out_hbm.at[pl.ds(c * NP_ + s * ROWS_PER_TILE, ROWS_PER_TILE)])


@functools.partial(
    pl.kernel,
    out_type=jax.ShapeDtypeStruct((2 * NP_, 128), jnp.float32),
    mesh=_mesh,
    scratch_types=[
        pltpu.VMEM((ICHUNK, 128), jnp.int32),
        pltpu.VMEM((ICHUNK, 128), jnp.int32),
        pltpu.VMEM((ICHUNK, 128), jnp.int32),
        pltpu.VMEM((ICHUNK, 128), jnp.int32),
        pltpu.VMEM((128, 128), jnp.float32),
        pltpu.VMEM((128, 128), jnp.float32),
        pltpu.VMEM_SHARED((NP_, 128), jnp.float32),
        pltpu.SemaphoreType.DMA,
        pltpu.SemaphoreType.DMA,
        pltpu.SemaphoreType.DMA,
        pltpu.SemaphoreType.DMA,
        pltpu.SemaphoreType.DMA,
    ],
)
def _sc_scatter(xws_hbm, src_hbm, dst_hbm, zeros_hbm, s_hbm,
                sidx0, didx0, sidx1, didx1, rows0, rows1, acc,
                semi, sem0, sem1, ssem0, ssem1):
    c = lax.axis_index("c")
    s = lax.axis_index("s")
    wid = c * NS + s
    rbase = wid * EROWS_PER_TILE
    sbufs = (sidx0, sidx1)
    dbufs = (didx0, didx1)

    pltpu.async_copy(src_hbm.at[pl.ds(rbase, ICHUNK)], sidx0, semi)
    pltpu.async_copy(dst_hbm.at[pl.ds(rbase, ICHUNK)], didx0, semi)
    pltpu.sync_copy(zeros_hbm, acc.at[pl.ds(s * ROWS_PER_TILE, ROWS_PER_TILE)])
    plsc.subcore_barrier()

    for t in range(NCHUNK):
        sA, dA = sbufs[t % 2], dbufs[t % 2]
        sB, dB = sbufs[(t + 1) % 2], dbufs[(t + 1) % 2]
        if t + 1 < NCHUNK:
            nb = rbase + (t + 1) * ICHUNK
            pltpu.async_copy(src_hbm.at[pl.ds(nb, ICHUNK)], sB, semi)
            pltpu.async_copy(dst_hbm.at[pl.ds(nb, ICHUNK)], dB, semi)
        cb = rbase + t * ICHUNK
        pltpu.make_async_copy(src_hbm.at[pl.ds(cb, ICHUNK)], sA, semi).wait()
        pltpu.make_async_copy(dst_hbm.at[pl.ds(cb, ICHUNK)], dA, semi).wait()

        pltpu.async_copy(xws_hbm.at[sA.at[0]], rows0, sem0)
        pltpu.async_copy(xws_hbm.at[sA.at[1]], rows1, sem1)

        def _body(j, carry, sA=sA, dA=dA):
            pltpu.make_async_copy(xws_hbm.at[sA.at[2 * j]], rows0, sem0).wait()
            pltpu.sync_copy(rows0, acc.at[dA.at[2 * j]], add=True)

            @pl.when(j < ICHUNK // 2 - 1)
            def _pf0():
                pltpu.async_copy(xws_hbm.at[sA.at[2 * j + 2]], rows0, sem0)

            pltpu.make_async_copy(xws_hbm.at[sA.at[2 * j + 1]], rows1, sem1).wait()
            pltpu.sync_copy(rows1, acc.at[dA.at[2 * j + 1]], add=True)

            @pl.when(j < ICHUNK // 2 - 1)
            def _pf1():
                pltpu.async_copy(xws_hbm.at[sA.at[2 * j + 3]], rows1, sem1)

            return carry

        lax.fori_loop(0, ICHUNK // 2, _body, 0)

    plsc.subcore_barrier()
    pltpu.sync_copy(acc.at[pl.ds(s * ROWS_PER_TILE, ROWS_PER_TILE)],
                    s_hbm.at[pl.ds(c * NP_ + s * ROWS_PER_TILE, ROWS_PER_TILE)])


def _gate_pre_body(batch_ref, x_ref, q_ref, w_ref, h_ref, xw_ref):
    gid = lax.broadcasted_iota(jnp.int32, (BLK, 128), 1)
    oh = (batch_ref[...] == gid).astype(jnp.float32)
    h = jnp.dot(oh, q_ref[...], preferred_element_type=jnp.float32) * x_ref[...]
    h_ref[...] = h
    xw_ref[...] = jnp.dot(h, w_ref[...], preferred_element_type=jnp.float32)


def _tc_gate_pre(batch_b, x, q_pad, w1):
    blk = pl.BlockSpec((BLK, 128), lambda i: (i, 0))
    full = pl.BlockSpec((128, 128), lambda i: (0, 0))
    out = jax.ShapeDtypeStruct((NP_, 128), jnp.float32)
    return pl.pallas_call(
        _gate_pre_body,
        grid=(NBLK,),
        in_specs=[blk, blk, full, full],
        out_specs=[blk, blk],
        out_shape=[out, out],
    )(batch_b, x, q_pad, w1)


def _gate_post_body(xw_ref, cnt_ref, xws_ref, dinv_ref):
    deg = cnt_ref[0, :, 0:1] + cnt_ref[1, :, 0:1] + 1.0
    dinv = jnp.broadcast_to(lax.rsqrt(deg), (BLK, 128))
    xws_ref[...] = xw_ref[...] * dinv
    dinv_ref[...] = dinv


def _tc_gate_post(xw1, counts):
    blk = pl.BlockSpec((BLK, 128), lambda i: (i, 0))
    cblk = pl.BlockSpec((2, BLK, 128), lambda i: (0, i, 0))
    out = jax.ShapeDtypeStruct((NP_, 128), jnp.float32)
    return pl.pallas_call(
        _gate_post_body,
        grid=(NBLK,),
        in_specs=[blk, cblk],
        out_specs=[blk, blk],
        out_shape=[out, out],
    )(xw1, counts)


def _mid_body(s_ref, xwp_ref, hp_ref, dinv_ref, b_ref, w_ref,
              h_ref, xw_ref, xws_ref):
    dv = dinv_ref[...]
    agg = dv * (s_ref[0] + s_ref[1]) + dv * dv * xwp_ref[...] + b_ref[...]
    h = jnp.maximum(agg + hp_ref[...], 0.0)
    xw = jnp.dot(h, w_ref[...], preferred_element_type=jnp.float32)
    h_ref[...] = h
    xw_ref[...] = xw
    xws_ref[...] = xw * dv


def _tc_mid(s_pair, xw_prev, h_prev, dinv_b, b_row, w_next):
    blk = pl.BlockSpec((BLK, 128), lambda i: (i, 0))
    sblk = pl.BlockSpec((2, BLK, 128), lambda i: (0, i, 0))
    brow = pl.BlockSpec((1, 128), lambda i: (0, 0))
    full = pl.BlockSpec((128, 128), lambda i: (0, 0))
    out = jax.ShapeDtypeStruct((NP_, 128), jnp.float32)
    return pl.pallas_call(
        _mid_body,
        grid=(NBLK,),
        in_specs=[sblk, blk, blk, blk, brow, full],
        out_specs=[blk, blk, blk],
        out_shape=[out, out, out],
    )(s_pair, xw_prev, h_prev, dinv_b, b_row, w_next)


FBLK = 1000


def _final_body(s_ref, xwp_ref, hp_ref, dinv_ref, b_ref, wc_ref, bc_ref,
                mask_ref, out_ref):
    dv = dinv_ref[...]
    agg = dv * (s_ref[0] + s_ref[1]) + dv * dv * xwp_ref[...] + b_ref[...]
    h4 = agg + hp_ref[...]
    logits = jnp.dot(h4, wc_ref[...], preferred_element_type=jnp.float32)
    out_ref[...] = (logits + bc_ref[...]) * mask_ref[...]


def _tc_final(s_pair, xw_prev, h_prev, dinv_b, b_row, wc, bc_row, mask):
    blk = pl.BlockSpec((FBLK, 128), lambda i: (i, 0))
    sblk = pl.BlockSpec((2, FBLK, 128), lambda i: (0, i, 0))
    brow = pl.BlockSpec((1, 128), lambda i: (0, 0))
    return pl.pallas_call(
        _final_body,
        grid=(N // FBLK,),
        in_specs=[sblk, blk, blk, blk, brow,
                  pl.BlockSpec((128, C), lambda i: (0, 0)),
                  pl.BlockSpec((1, C), lambda i: (0, 0)),
                  pl.BlockSpec((FBLK, C), lambda i: (i, 0))],
        out_specs=pl.BlockSpec((FBLK, C), lambda i: (i, 0)),
        out_shape=jax.ShapeDtypeStruct((N, C), jnp.float32),
    )(s_pair, xw_prev, h_prev, dinv_b, b_row, wc, bc_row, mask)


def kernel(x, query, batch, edge_index, W1, b1, W2, b2, W3, b3, W4, b4, Wc, bc):
    f32 = jnp.float32
    x_p = jnp.pad(x, ((0, NP_ - N), (0, 0)))
    batch_p = jnp.pad(batch, (0, NP_ - N))
    batch_b = jnp.broadcast_to(batch_p[:, None], (NP_, 128))
    q_pad = jnp.pad(query, ((0, 128 - G), (0, 0)))
    bc_row = bc.reshape(1, C)
    b1r, b2r, b3r, b4r = (b.reshape(1, 128) for b in (b1, b2, b3, b4))

    pad_idx = N + jnp.arange(EP - E, dtype=jnp.int32) % (NP_ - N)
    src = jnp.concatenate([edge_index[0], pad_idx]).reshape(EROWS, 128)
    dst = jnp.concatenate([edge_index[1], pad_idx]).reshape(EROWS, 128)
    zeros = jnp.zeros((ROWS_PER_TILE, 128), f32)

    keep = jax.random.bernoulli(jax.random.key(42), 1.0 - P, (N, C))
    maskf = jnp.where(keep, f32(1.0) / f32(1.0 - P), f32(0.0))

    ones128 = jnp.ones((128, 128), f32)
    counts = _sc_degree(dst, ones128, zeros).reshape(2, NP_, 128)

    h0, xw1 = _tc_gate_pre(batch_b, x_p, q_pad, W1)
    xws1, dinv_b = _tc_gate_post(xw1, counts)
    s1 = _sc_scatter(xws1, src, dst, zeros).reshape(2, NP_, 128)
    h1, xw2, xws2 = _tc_mid(s1, xw1, h0, dinv_b, b1r, W2)
    s2 = _sc_scatter(xws2, src, dst, zeros).reshape(2, NP_, 128)
    h2, xw3, xws3 = _tc_mid(s2, xw2, h1, dinv_b, b2r, W3)
    s3 = _sc_scatter(xws3, src, dst, zeros).reshape(2, NP_, 128)
    h3, xw4, xws4 = _tc_mid(s3, xw3, h2, dinv_b, b3r, W4)
    s4 = _sc_scatter(xws4, src, dst, zeros).reshape(2, NP_, 128)
    return _tc_final(s4, xw4, h3, dinv_b, b4r, Wc, bc_row, maskf)


# speedup vs baseline: 1.3403x; 1.0124x over previous
import functools

import jax
import jax.numpy as jnp
from jax import lax
from jax.experimental import pallas as pl
from jax.experimental.pallas import tpu as pltpu
from jax.experimental.pallas import tpu_sc as plsc

N = 10000
E = 320000
D = 128
H = 128
C = 40
G = 16
P = 0.2

NP_ = 10240
EP = 327680
NC = 2
NS = 16
NW = NC * NS
ROWS_PER_TILE = NP_ // NS
EROWS = EP // 128
EROWS_PER_TILE = EROWS // NW
ICHUNK = 16
NCHUNK = EROWS_PER_TILE // ICHUNK
BLK = 2048
NBLK = NP_ // BLK

_mesh = plsc.VectorSubcoreMesh(core_axis_name="c", subcore_axis_name="s")


@functools.partial(
    pl.kernel,
    out_type=jax.ShapeDtypeStruct((2 * NP_, 128), jnp.float32),
    mesh=_mesh,
    scratch_types=[
        pltpu.VMEM((EROWS_PER_TILE, 128), jnp.int32),
        pltpu.VMEM((128, 128), jnp.float32),
        pltpu.VMEM_SHARED((NP_, 128), jnp.float32),
        pltpu.SemaphoreType.DMA,
        pltpu.SemaphoreType.DMA,
    ],
)
def _sc_degree(dst_hbm, ones_hbm, zeros_hbm, out_hbm, didx, ones_v, cnt,
               sem0, sem1):
    c = lax.axis_index("c")
    s = lax.axis_index("s")
    wid = c * NS + s
    pltpu.sync_copy(dst_hbm.at[pl.ds(wid * EROWS_PER_TILE, EROWS_PER_TILE)], didx)
    pltpu.sync_copy(ones_hbm, ones_v)
    pltpu.sync_copy(zeros_hbm, cnt.at[pl.ds(s * ROWS_PER_TILE, ROWS_PER_TILE)])
    plsc.subcore_barrier()

    def _body(j, _):
        ds_ = [pltpu.async_copy(ones_v, cnt.at[didx.at[8 * j + k]],
                                sem0 if k % 2 == 0 else sem1, add=True)
               for k in range(8)]
        for d in ds_:
            d.wait()
        return _

    lax.fori_loop(0, EROWS_PER_TILE // 8, _body, 0)
    plsc.subcore_barrier()
    pltpu.sync_copy(cnt.at[pl.ds(s * ROWS_PER_TILE, ROWS_PER_TILE)],
                    out_hbm.at[pl.ds(c * NP_ + s * ROWS_PER_TILE, ROWS_PER_TILE)])


@functools.partial(
    pl.kernel,
    out_type=jax.ShapeDtypeStruct((2 * NP_, 128), jnp.float32),
    mesh=_mesh,
    scratch_types=[
        pltpu.VMEM((ICHUNK, 128), jnp.int32),
        pltpu.VMEM((ICHUNK, 128), jnp.int32),
        pltpu.VMEM((ICHUNK, 128), jnp.int32),
        pltpu.VMEM((ICHUNK, 128), jnp.int32),
        pltpu.VMEM((128, 128), jnp.float32),
        pltpu.VMEM((128, 128), jnp.float32),
        pltpu.VMEM_SHARED((NP_, 128), jnp.float32),
        pltpu.SemaphoreType.DMA,
        pltpu.SemaphoreType.DMA,
        pltpu.SemaphoreType.DMA,
        pltpu.SemaphoreType.DMA,
        pltpu.SemaphoreType.DMA,
    ],
)
def _sc_scatter(xws_hbm, src_hbm, dst_hbm, zeros_hbm, s_hbm,
                sidx0, didx0, sidx1, didx1, rows0, rows1, acc,
                semi, sem0, sem1, ssem0, ssem1):
    c = lax.axis_index("c")
    s = lax.axis_index("s")
    wid = c * NS + s
    rbase = wid * EROWS_PER_TILE
    sbufs = (sidx0, sidx1)
    dbufs = (didx0, didx1)

    pltpu.async_copy(src_hbm.at[pl.ds(rbase, ICHUNK)], sidx0, semi)
    pltpu.async_copy(dst_hbm.at[pl.ds(rbase, ICHUNK)], didx0, semi)
    pltpu.sync_copy(zeros_hbm, acc.at[pl.ds(s * ROWS_PER_TILE, ROWS_PER_TILE)])
    plsc.subcore_barrier()

    for t in range(NCHUNK):
        sA, dA = sbufs[t % 2], dbufs[t % 2]
        sB, dB = sbufs[(t + 1) % 2], dbufs[(t + 1) % 2]
        if t + 1 < NCHUNK:
            nb = rbase + (t + 1) * ICHUNK
            pltpu.async_copy(src_hbm.at[pl.ds(nb, ICHUNK)], sB, semi)
            pltpu.async_copy(dst_hbm.at[pl.ds(nb, ICHUNK)], dB, semi)
        cb = rbase + t * ICHUNK
        pltpu.make_async_copy(src_hbm.at[pl.ds(cb, ICHUNK)], sA, semi).wait()
        pltpu.make_async_copy(dst_hbm.at[pl.ds(cb, ICHUNK)], dA, semi).wait()

        pltpu.async_copy(xws_hbm.at[sA.at[0]], rows0, sem0)
        pltpu.async_copy(xws_hbm.at[sA.at[1]], rows1, sem1)

        def _body(j, carry, sA=sA, dA=dA):
            pltpu.make_async_copy(xws_hbm.at[sA.at[2 * j]], rows0, sem0).wait()
            pltpu.sync_copy(rows0, acc.at[dA.at[2 * j]], add=True)

            @pl.when(j < ICHUNK // 2 - 1)
            def _pf0():
                pltpu.async_copy(xws_hbm.at[sA.at[2 * j + 2]], rows0, sem0)

            pltpu.make_async_copy(xws_hbm.at[sA.at[2 * j + 1]], rows1, sem1).wait()
            pltpu.sync_copy(rows1, acc.at[dA.at[2 * j + 1]], add=True)

            @pl.when(j < ICHUNK // 2 - 1)
            def _pf1():
                pltpu.async_copy(xws_hbm.at[sA.at[2 * j + 3]], rows1, sem1)

            return carry

        lax.fori_loop(0, ICHUNK // 2, _body, 0)

    plsc.subcore_barrier()
    pltpu.sync_copy(acc.at[pl.ds(s * ROWS_PER_TILE, ROWS_PER_TILE)],
                    s_hbm.at[pl.ds(c * NP_ + s * ROWS_PER_TILE, ROWS_PER_TILE)])


def _gate_pre_body(batch_ref, x_ref, q_ref, w_ref, h_ref, xw_ref):
    gid = lax.broadcasted_iota(jnp.int32, (BLK, 128), 1)
    oh = (batch_ref[...] == gid).astype(jnp.float32)
    h = jnp.dot(oh, q_ref[...], preferred_element_type=jnp.float32) * x_ref[...]
    h_ref[...] = h
    xw_ref[...] = jnp.dot(h, w_ref[...], preferred_element_type=jnp.float32)


def _tc_gate_pre(batch_b, x, q_pad, w1):
    blk = pl.BlockSpec((BLK, 128), lambda i: (i, 0))
    full = pl.BlockSpec((128, 128), lambda i: (0, 0))
    out = jax.ShapeDtypeStruct((NP_, 128), jnp.float32)
    return pl.pallas_call(
        _gate_pre_body,
        grid=(NBLK,),
        in_specs=[blk, blk, full, full],
        out_specs=[blk, blk],
        out_shape=[out, out],
    )(batch_b, x, q_pad, w1)


def _gate_post_body(xw_ref, cnt_ref, xws_ref, dinv_ref):
    deg = cnt_ref[0, :, 0:1] + cnt_ref[1, :, 0:1] + 1.0
    dinv = jnp.broadcast_to(lax.rsqrt(deg), (BLK, 128))
    xws_ref[...] = xw_ref[...] * dinv
    dinv_ref[...] = dinv


def _tc_gate_post(xw1, counts):
    blk = pl.BlockSpec((BLK, 128), lambda i: (i, 0))
    cblk = pl.BlockSpec((2, BLK, 128), lambda i: (0, i, 0))
    out = jax.ShapeDtypeStruct((NP_, 128), jnp.float32)
    return pl.pallas_call(
        _gate_post_body,
        grid=(NBLK,),
        in_specs=[blk, cblk],
        out_specs=[blk, blk],
        out_shape=[out, out],
    )(xw1, counts)


def _mid_body(s_ref, xwp_ref, hp_ref, dinv_ref, b_ref, w_ref,
              h_ref, xw_ref, xws_ref):
    dv = dinv_ref[...]
    agg = dv * (s_ref[0] + s_ref[1]) + dv * dv * xwp_ref[...] + b_ref[...]
    h = jnp.maximum(agg + hp_ref[...], 0.0)
    xw = jnp.dot(h, w_ref[...], preferred_element_type=jnp.float32)
    h_ref[...] = h
    xw_ref[...] = xw
    xws_ref[...] = xw * dv


def _tc_mid(s_pair, xw_prev, h_prev, dinv_b, b_row, w_next):
    blk = pl.BlockSpec((BLK, 128), lambda i: (i, 0))
    sblk = pl.BlockSpec((2, BLK, 128), lambda i: (0, i, 0))
    brow = pl.BlockSpec((1, 128), lambda i: (0, 0))
    full = pl.BlockSpec((128, 128), lambda i: (0, 0))
    out = jax.ShapeDtypeStruct((NP_, 128), jnp.float32)
    return pl.pallas_call(
        _mid_body,
        grid=(NBLK,),
        in_specs=[sblk, blk, blk, blk, brow, full],
        out_specs=[blk, blk, blk],
        out_shape=[out, out, out],
    )(s_pair, xw_prev, h_prev, dinv_b, b_row, w_next)


FBLK = 2000


def _final_body(s_ref, xwp_ref, hp_ref, dinv_ref, b_ref, wc_ref, bc_ref,
                mask_ref, out_ref):
    dv = dinv_ref[...]
    agg = dv * (s_ref[0] + s_ref[1]) + dv * dv * xwp_ref[...] + b_ref[...]
    h4 = agg + hp_ref[...]
    logits = jnp.dot(h4, wc_ref[...], preferred_element_type=jnp.float32)
    out_ref[...] = (logits + bc_ref[...]) * mask_ref[...]


def _tc_final(s_pair, xw_prev, h_prev, dinv_b, b_row, wc, bc_row, mask):
    blk = pl.BlockSpec((FBLK, 128), lambda i: (i, 0))
    sblk = pl.BlockSpec((2, FBLK, 128), lambda i: (0, i, 0))
    brow = pl.BlockSpec((1, 128), lambda i: (0, 0))
    return pl.pallas_call(
        _final_body,
        grid=(N // FBLK,),
        in_specs=[sblk, blk, blk, blk, brow,
                  pl.BlockSpec((128, C), lambda i: (0, 0)),
                  pl.BlockSpec((1, C), lambda i: (0, 0)),
                  pl.BlockSpec((FBLK, C), lambda i: (i, 0))],
        out_specs=pl.BlockSpec((FBLK, C), lambda i: (i, 0)),
        out_shape=jax.ShapeDtypeStruct((N, C), jnp.float32),
    )(s_pair, xw_prev, h_prev, dinv_b, b_row, wc, bc_row, mask)


def kernel(x, query, batch, edge_index, W1, b1, W2, b2, W3, b3, W4, b4, Wc, bc):
    f32 = jnp.float32
    x_p = jnp.pad(x, ((0, NP_ - N), (0, 0)))
    batch_p = jnp.pad(batch, (0, NP_ - N))
    batch_b = jnp.broadcast_to(batch_p[:, None], (NP_, 128))
    q_pad = jnp.pad(query, ((0, 128 - G), (0, 0)))
    bc_row = bc.reshape(1, C)
    b1r, b2r, b3r, b4r = (b.reshape(1, 128) for b in (b1, b2, b3, b4))

    pad_idx = N + jnp.arange(EP - E, dtype=jnp.int32) % (NP_ - N)
    src = jnp.concatenate([edge_index[0], pad_idx]).reshape(EROWS, 128)
    dst = jnp.concatenate([edge_index[1], pad_idx]).reshape(EROWS, 128)
    zeros = jnp.zeros((ROWS_PER_TILE, 128), f32)

    keep = jax.random.bernoulli(jax.random.key(42), 1.0 - P, (N, C))
    maskf = jnp.where(keep, f32(1.0) / f32(1.0 - P), f32(0.0))

    ones128 = jnp.ones((128, 128), f32)
    counts = _sc_degree(dst, ones128, zeros).reshape(2, NP_, 128)

    h0, xw1 = _tc_gate_pre(batch_b, x_p, q_pad, W1)
    xws1, dinv_b = _tc_gate_post(xw1, counts)
    s1 = _sc_scatter(xws1, src, dst, zeros).reshape(2, NP_, 128)
    h1, xw2, xws2 = _tc_mid(s1, xw1, h0, dinv_b, b1r, W2)
    s2 = _sc_scatter(xws2, src, dst, zeros).reshape(2, NP_, 128)
    h2, xw3, xws3 = _tc_mid(s2, xw2, h1, dinv_b, b2r, W3)
    s3 = _sc_scatter(xws3, src, dst, zeros).reshape(2, NP_, 128)
    h3, xw4, xws4 = _tc_mid(s3, xw3, h2, dinv_b, b3r, W4)
    s4 = _sc_scatter(xws4, src, dst, zeros).reshape(2, NP_, 128)
    return _tc_final(s4, xw4, h3, dinv_b, b4r, Wc, bc_row, maskf)


# speedup vs baseline: 1.3417x; 1.0011x over previous
import functools

import jax
import jax.numpy as jnp
from jax import lax
from jax.experimental import pallas as pl
from jax.experimental.pallas import tpu as pltpu
from jax.experimental.pallas import tpu_sc as plsc

N = 10000
E = 320000
D = 128
H = 128
C = 40
G = 16
P = 0.2

NP_ = 10240
EP = 327680
NC = 2
NS = 16
NW = NC * NS
ROWS_PER_TILE = NP_ // NS
EROWS = EP // 128
EROWS_PER_TILE = EROWS // NW
ICHUNK = 16
NCHUNK = EROWS_PER_TILE // ICHUNK
BLK = 2560
NBLK = NP_ // BLK

_mesh = plsc.VectorSubcoreMesh(core_axis_name="c", subcore_axis_name="s")


@functools.partial(
    pl.kernel,
    out_type=jax.ShapeDtypeStruct((2 * NP_, 128), jnp.float32),
    mesh=_mesh,
    scratch_types=[
        pltpu.VMEM((EROWS_PER_TILE, 128), jnp.int32),
        pltpu.VMEM((128, 128), jnp.float32),
        pltpu.VMEM_SHARED((NP_, 128), jnp.float32),
        pltpu.SemaphoreType.DMA,
        pltpu.SemaphoreType.DMA,
    ],
)
def _sc_degree(dst_hbm, ones_hbm, zeros_hbm, out_hbm, didx, ones_v, cnt,
               sem0, sem1):
    c = lax.axis_index("c")
    s = lax.axis_index("s")
    wid = c * NS + s
    pltpu.sync_copy(dst_hbm.at[pl.ds(wid * EROWS_PER_TILE, EROWS_PER_TILE)], didx)
    pltpu.sync_copy(ones_hbm, ones_v)
    pltpu.sync_copy(zeros_hbm, cnt.at[pl.ds(s * ROWS_PER_TILE, ROWS_PER_TILE)])
    plsc.subcore_barrier()

    def _body(j, _):
        ds_ = [pltpu.async_copy(ones_v, cnt.at[didx.at[8 * j + k]],
                                sem0 if k % 2 == 0 else sem1, add=True)
               for k in range(8)]
        for d in ds_:
            d.wait()
        return _

    lax.fori_loop(0, EROWS_PER_TILE // 8, _body, 0)
    plsc.subcore_barrier()
    pltpu.sync_copy(cnt.at[pl.ds(s * ROWS_PER_TILE, ROWS_PER_TILE)],
                    out_hbm.at[pl.ds(c * NP_ + s * ROWS_PER_TILE, ROWS_PER_TILE)])


@functools.partial(
    pl.kernel,
    out_type=jax.ShapeDtypeStruct((2 * NP_, 128), jnp.float32),
    mesh=_mesh,
    scratch_types=[
        pltpu.VMEM((ICHUNK, 128), jnp.int32),
        pltpu.VMEM((ICHUNK, 128), jnp.int32),
        pltpu.VMEM((ICHUNK, 128), jnp.int32),
        pltpu.VMEM((ICHUNK, 128), jnp.int32),
        pltpu.VMEM((128, 128), jnp.float32),
        pltpu.VMEM((128, 128), jnp.float32),
        pltpu.VMEM_SHARED((NP_, 128), jnp.float32),
        pltpu.SemaphoreType.DMA,
        pltpu.SemaphoreType.DMA,
        pltpu.SemaphoreType.DMA,
        pltpu.SemaphoreType.DMA,
        pltpu.SemaphoreType.DMA,
    ],
)
def _sc_scatter(xws_hbm, src_hbm, dst_hbm, zeros_hbm, s_hbm,
                sidx0, didx0, sidx1, didx1, rows0, rows1, acc,
                semi, sem0, sem1, ssem0, ssem1):
    c = lax.axis_index("c")
    s = lax.axis_index("s")
    wid = c * NS + s
    rbase = wid * EROWS_PER_TILE
    sbufs = (sidx0, sidx1)
    dbufs = (didx0, didx1)

    pltpu.async_copy(src_hbm.at[pl.ds(rbase, ICHUNK)], sidx0, semi)
    pltpu.async_copy(dst_hbm.at[pl.ds(rbase, ICHUNK)], didx0, semi)
    pltpu.sync_copy(zeros_hbm, acc.at[pl.ds(s * ROWS_PER_TILE, ROWS_PER_TILE)])
    plsc.subcore_barrier()

    for t in range(NCHUNK):
        sA, dA = sbufs[t % 2], dbufs[t % 2]
        sB, dB = sbufs[(t + 1) % 2], dbufs[(t + 1) % 2]
        if t + 1 < NCHUNK:
            nb = rbase + (t + 1) * ICHUNK
            pltpu.async_copy(src_hbm.at[pl.ds(nb, ICHUNK)], sB, semi)
            pltpu.async_copy(dst_hbm.at[pl.ds(nb, ICHUNK)], dB, semi)
        cb = rbase + t * ICHUNK
        pltpu.make_async_copy(src_hbm.at[pl.ds(cb, ICHUNK)], sA, semi).wait()
        pltpu.make_async_copy(dst_hbm.at[pl.ds(cb, ICHUNK)], dA, semi).wait()

        pltpu.async_copy(xws_hbm.at[sA.at[0]], rows0, sem0)
        pltpu.async_copy(xws_hbm.at[sA.at[1]], rows1, sem1)

        def _body(j, carry, sA=sA, dA=dA):
            pltpu.make_async_copy(xws_hbm.at[sA.at[2 * j]], rows0, sem0).wait()
            pltpu.sync_copy(rows0, acc.at[dA.at[2 * j]], add=True)

            @pl.when(j < ICHUNK // 2 - 1)
            def _pf0():
                pltpu.async_copy(xws_hbm.at[sA.at[2 * j + 2]], rows0, sem0)

            pltpu.make_async_copy(xws_hbm.at[sA.at[2 * j + 1]], rows1, sem1).wait()
            pltpu.sync_copy(rows1, acc.at[dA.at[2 * j + 1]], add=True)

            @pl.when(j < ICHUNK // 2 - 1)
            def _pf1():
                pltpu.async_copy(xws_hbm.at[sA.at[2 * j + 3]], rows1, sem1)

            return carry

        lax.fori_loop(0, ICHUNK // 2, _body, 0)

    plsc.subcore_barrier()
    pltpu.sync_copy(acc.at[pl.ds(s * ROWS_PER_TILE, ROWS_PER_TILE)],
                    s_hbm.at[pl.ds(c * NP_ + s * ROWS_PER_TILE, ROWS_PER_TILE)])


def _gate_pre_body(batch_ref, x_ref, q_ref, w_ref, h_ref, xw_ref):
    gid = lax.broadcasted_iota(jnp.int32, (BLK, 128), 1)
    oh = (batch_ref[...] == gid).astype(jnp.float32)
    h = jnp.dot(oh, q_ref[...], preferred_element_type=jnp.float32) * x_ref[...]
    h_ref[...] = h
    xw_ref[...] = jnp.dot(h, w_ref[...], preferred_element_type=jnp.float32)


def _tc_gate_pre(batch_b, x, q_pad, w1):
    blk = pl.BlockSpec((BLK, 128), lambda i: (i, 0))
    full = pl.BlockSpec((128, 128), lambda i: (0, 0))
    out = jax.ShapeDtypeStruct((NP_, 128), jnp.float32)
    return pl.pallas_call(
        _gate_pre_body,
        grid=(NBLK,),
        in_specs=[blk, blk, full, full],
        out_specs=[blk, blk],
        out_shape=[out, out],
    )(batch_b, x, q_pad, w1)


def _gate_post_body(xw_ref, cnt_ref, xws_ref, dinv_ref):
    deg = cnt_ref[0, :, 0:1] + cnt_ref[1, :, 0:1] + 1.0
    dinv = jnp.broadcast_to(lax.rsqrt(deg), (BLK, 128))
    xws_ref[...] = xw_ref[...] * dinv
    dinv_ref[...] = dinv


def _tc_gate_post(xw1, counts):
    blk = pl.BlockSpec((BLK, 128), lambda i: (i, 0))
    cblk = pl.BlockSpec((2, BLK, 128), lambda i: (0, i, 0))
    out = jax.ShapeDtypeStruct((NP_, 128), jnp.float32)
    return pl.pallas_call(
        _gate_post_body,
        grid=(NBLK,),
        in_specs=[blk, cblk],
        out_specs=[blk, blk],
        out_shape=[out, out],
    )(xw1, counts)


def _mid_body(s_ref, xwp_ref, hp_ref, dinv_ref, b_ref, w_ref,
              h_ref, xw_ref, xws_ref):
    dv = dinv_ref[...]
    agg = dv * (s_ref[0] + s_ref[1]) + dv * dv * xwp_ref[...] + b_ref[...]
    h = jnp.maximum(agg + hp_ref[...], 0.0)
    xw = jnp.dot(h, w_ref[...], preferred_element_type=jnp.float32)
    h_ref[...] = h
    xw_ref[...] = xw
    xws_ref[...] = xw * dv


def _tc_mid(s_pair, xw_prev, h_prev, dinv_b, b_row, w_next):
    blk = pl.BlockSpec((BLK, 128), lambda i: (i, 0))
    sblk = pl.BlockSpec((2, BLK, 128), lambda i: (0, i, 0))
    brow = pl.BlockSpec((1, 128), lambda i: (0, 0))
    full = pl.BlockSpec((128, 128), lambda i: (0, 0))
    out = jax.ShapeDtypeStruct((NP_, 128), jnp.float32)
    return pl.pallas_call(
        _mid_body,
        grid=(NBLK,),
        in_specs=[sblk, blk, blk, blk, brow, full],
        out_specs=[blk, blk, blk],
        out_shape=[out, out, out],
    )(s_pair, xw_prev, h_prev, dinv_b, b_row, w_next)


FBLK = 2000


def _final_body(s_ref, xwp_ref, hp_ref, dinv_ref, b_ref, wc_ref, bc_ref,
                mask_ref, out_ref):
    dv = dinv_ref[...]
    agg = dv * (s_ref[0] + s_ref[1]) + dv * dv * xwp_ref[...] + b_ref[...]
    h4 = agg + hp_ref[...]
    logits = jnp.dot(h4, wc_ref[...], preferred_element_type=jnp.float32)
    out_ref[...] = (logits + bc_ref[...]) * mask_ref[...]


def _tc_final(s_pair, xw_prev, h_prev, dinv_b, b_row, wc, bc_row, mask):
    blk = pl.BlockSpec((FBLK, 128), lambda i: (i, 0))
    sblk = pl.BlockSpec((2, FBLK, 128), lambda i: (0, i, 0))
    brow = pl.BlockSpec((1, 128), lambda i: (0, 0))
    return pl.pallas_call(
        _final_body,
        grid=(N // FBLK,),
        in_specs=[sblk, blk, blk, blk, brow,
                  pl.BlockSpec((128, C), lambda i: (0, 0)),
                  pl.BlockSpec((1, C), lambda i: (0, 0)),
                  pl.BlockSpec((FBLK, C), lambda i: (i, 0))],
        out_specs=pl.BlockSpec((FBLK, C), lambda i: (i, 0)),
        out_shape=jax.ShapeDtypeStruct((N, C), jnp.float32),
    )(s_pair, xw_prev, h_prev, dinv_b, b_row, wc, bc_row, mask)


def kernel(x, query, batch, edge_index, W1, b1, W2, b2, W3, b3, W4, b4, Wc, bc):
    f32 = jnp.float32
    x_p = jnp.pad(x, ((0, NP_ - N), (0, 0)))
    batch_p = jnp.pad(batch, (0, NP_ - N))
    batch_b = jnp.broadcast_to(batch_p[:, None], (NP_, 128))
    q_pad = jnp.pad(query, ((0, 128 - G), (0, 0)))
    bc_row = bc.reshape(1, C)
    b1r, b2r, b3r, b4r = (b.reshape(1, 128) for b in (b1, b2, b3, b4))

    pad_idx = N + jnp.arange(EP - E, dtype=jnp.int32) % (NP_ - N)
    src = jnp.concatenate([edge_index[0], pad_idx]).reshape(EROWS, 128)
    dst = jnp.concatenate([edge_index[1], pad_idx]).reshape(EROWS, 128)
    zeros = jnp.zeros((ROWS_PER_TILE, 128), f32)

    keep = jax.random.bernoulli(jax.random.key(42), 1.0 - P, (N, C))
    maskf = jnp.where(keep, f32(1.0) / f32(1.0 - P), f32(0.0))

    ones128 = jnp.ones((128, 128), f32)
    counts = _sc_degree(dst, ones128, zeros).reshape(2, NP_, 128)

    h0, xw1 = _tc_gate_pre(batch_b, x_p, q_pad, W1)
    xws1, dinv_b = _tc_gate_post(xw1, counts)
    s1 = _sc_scatter(xws1, src, dst, zeros).reshape(2, NP_, 128)
    h1, xw2, xws2 = _tc_mid(s1, xw1, h0, dinv_b, b1r, W2)
    s2 = _sc_scatter(xws2, src, dst, zeros).reshape(2, NP_, 128)
    h2, xw3, xws3 = _tc_mid(s2, xw2, h1, dinv_b, b2r, W3)
    s3 = _sc_scatter(xws3, src, dst, zeros).reshape(2, NP_, 128)
    h3, xw4, xws4 = _tc_mid(s3, xw3, h2, dinv_b, b3r, W4)
    s4 = _sc_scatter(xws4, src, dst, zeros).reshape(2, NP_, 128)
    return _tc_final(s4, xw4, h3, dinv_b, b4r, Wc, bc_row, maskf)
